# baseline (speedup 1.0000x reference)
import functools

import jax
import jax.numpy as jnp
import numpy as np
from jax import lax
from jax.experimental import pallas as pl
from jax.experimental.pallas import tpu as pltpu
from jax.experimental.pallas import tpu_sc as plsc

_SHIFT = 8192
_SOURCES = 2
_BATCH = 16
_LENGTH = 163840
_CHANNELS = 2
_OUT_LEN = _LENGTH - _SHIFT

_NC = 2
_NS = 16
_NW = _NC * _NS

_LANES = 128
_IN_T = _LENGTH // _LANES
_OUT_T = _OUT_LEN // _LANES
_IN_R = 2 * _IN_T
_OUT_R = 2 * _OUT_T
_CTILES = 76
_CROWS = 2 * _CTILES
_CHUNK = _CROWS * _LANES
_NCHUNK = _OUT_T // _CTILES
_IBROWS = _CROWS + 16


def _rotl(x, d):
    return ((x << np.uint32(d)) | (x >> np.uint32(32 - d))).astype(np.uint32)


def _threefry2x32_pair(key, x0, x1):
    x0 = x0.astype(np.uint32).copy()
    x1 = x1.astype(np.uint32).copy()
    ks = [np.uint32(key[0]), np.uint32(key[1])]
    ks.append(np.uint32(ks[0] ^ ks[1] ^ np.uint32(0x1BD11BDA)))
    x0 = (x0 + ks[0]).astype(np.uint32)
    x1 = (x1 + ks[1]).astype(np.uint32)
    rotations = [[13, 15, 26, 6], [17, 29, 16, 24]]
    for i in range(5):
        for r in rotations[i % 2]:
            x0 = (x0 + x1).astype(np.uint32)
            x1 = _rotl(x1, r)
            x1 = x1 ^ x0
        x0 = (x0 + ks[(i + 1) % 3]).astype(np.uint32)
        x1 = (x1 + ks[(i + 2) % 3] + np.uint32(i + 1)).astype(np.uint32)
    return x0, x1


def _get_offsets() -> np.ndarray:
    key = np.array([0, 42], dtype=np.uint32)
    b1, b2 = _threefry2x32_pair(
        key, np.zeros(2, np.uint32), np.arange(2, dtype=np.uint32)
    )
    k2 = np.array([b1[1], b2[1]], dtype=np.uint32)
    c1, c2 = _threefry2x32_pair(
        k2, np.zeros(32, np.uint32), np.arange(32, dtype=np.uint32)
    )
    bits = (c1 ^ c2).reshape(_SOURCES, _BATCH)
    return (bits % np.uint32(_SHIFT)).astype(np.int32)


def _select_const(wid, values):
    acc = jnp.int32(values[0])
    for w in range(1, len(values)):
        acc = lax.select(wid == w, jnp.int32(values[w]), acc)
    return acc


def _make_shift_kernel(offs):
    mesh = plsc.VectorSubcoreMesh(
        core_axis_name="c", subcore_axis_name="s", num_cores=_NC
    )
    scratch = [
        pltpu.VMEM((_IBROWS, _LANES), jnp.float32),
        pltpu.VMEM((_IBROWS, _LANES), jnp.float32),
        pltpu.VMEM((_CROWS, _LANES), jnp.float32),
        pltpu.VMEM((_CROWS, _LANES), jnp.float32),
    ]
    scratch += [pltpu.SemaphoreType.DMA] * 4

    arows = [
        w * _IN_R + 2 * ((o // _LANES) - (o // _LANES) % 4)
        for w, o in enumerate(offs)
    ]
    mqs = [(o // _LANES) % 4 for o in offs]
    rs = [o % _LANES for o in offs]

    @functools.partial(
        pl.kernel,
        out_type=jax.ShapeDtypeStruct((_NW * _OUT_R, _LANES), jnp.float32),
        mesh=mesh,
        scratch_types=scratch,
        compiler_params=pltpu.CompilerParams(needs_layout_passes=False),
    )
    def shift_kernel(wav_hbm, out_hbm, ib0, ib1, ob0, ob1, si0, si1, so0, so1):
        wav2d = wav_hbm
        out2d = out_hbm
        ibufs, obufs = [ib0, ib1], [ob0, ob1]
        in_sems, out_sems = [si0, si1], [so0, so1]
        wid = lax.axis_index("s") * _NC + lax.axis_index("c")
        arow = pl.multiple_of(_select_const(wid, arows), 8)
        obrow = pl.multiple_of(wid * _OUT_R, 8)
        mq = _select_const(wid, mqs)
        r = _select_const(wid, rs)
        lane = jnp.arange(16, dtype=jnp.int32)
        rvec = lane + r

        def in_copy(k):
            return pltpu.make_async_copy(
                wav2d.at[pl.ds(arow + k * _CROWS, _IBROWS), :],
                ibufs[k % 2],
                in_sems[k % 2],
            )

        def out_copy(k):
            return pltpu.make_async_copy(
                obufs[k % 2],
                out2d.at[pl.ds(obrow + k * _CROWS, _CROWS), :],
                out_sems[k % 2],
            )

        carry2s = []
        cols = []
        for u in range(8):
            svec = rvec + u * 16
            carry2s.append((svec >> 7) << 1)
            cols.append(svec & 127)

        def shift_chunk(k):
            ib, ob = ibufs[k % 2], obufs[k % 2]

            @plsc.parallel_loop(0, _CHUNK // 16, step=8, unroll=2)
            def body(w0):
                j = w0 >> 4
                c = (w0 >> 3) & 1
                rowbase = 2 * (j + mq) + c
                orow = w0 >> 3
                for u in range(8):
                    vec = plsc.load_gather(
                        ib, [rowbase + carry2s[u], cols[u]]
                    )
                    ob[orow, pl.ds(u * 16, 16)] = vec

        in_copy(0).start()
        in_copy(1).start()
        for k in range(_NCHUNK):
            in_copy(k).wait()
            if k >= 2:
                out_copy(k - 2).wait()
            shift_chunk(k)
            out_copy(k).start()
            if k + 2 < _NCHUNK:
                in_copy(k + 2).start()
        out_copy(_NCHUNK - 2).wait()
        out_copy(_NCHUNK - 1).wait()

    return shift_kernel


def kernel(wav):
    offs_np = _get_offsets()
    offs = [int(offs_np[w // _BATCH, w % _BATCH]) for w in range(_NW)]
    wav2 = (
        wav.reshape(_SOURCES, _BATCH, _IN_T, _LANES, _CHANNELS)
        .transpose(0, 1, 2, 4, 3)
        .reshape(_NW * _IN_R, _LANES)
    )
    out2 = _make_shift_kernel(offs)(wav2)
    out = (
        out2.reshape(_SOURCES, _BATCH, _OUT_T, _CHANNELS, _LANES)
        .transpose(0, 1, 2, 4, 3)
        .reshape(_SOURCES, _BATCH, _OUT_LEN, _CHANNELS)
    )
    offsets = jnp.asarray(offs_np)
    return (out, offsets)


# speedup vs baseline: 1.0464x; 1.0464x over previous
import functools

import jax
import jax.numpy as jnp
import numpy as np
from jax import lax
from jax.experimental import pallas as pl
from jax.experimental.pallas import tpu as pltpu
from jax.experimental.pallas import tpu_sc as plsc

_SHIFT = 8192
_SOURCES = 2
_BATCH = 16
_LENGTH = 163840
_CHANNELS = 2
_OUT_LEN = _LENGTH - _SHIFT

_NC = 2
_NS = 16
_NW = _NC * _NS

_LANES = 128
_IN_T = _LENGTH // _LANES
_OUT_T = _OUT_LEN // _LANES
_IN_R = 2 * _IN_T
_OUT_R = 2 * _OUT_T
_CTILES = 76
_CROWS = 2 * _CTILES
_CHUNK = _CROWS * _LANES
_NCHUNK = _OUT_T // _CTILES
_IBROWS = _CROWS + 16


def _rotl(x, d):
    return ((x << np.uint32(d)) | (x >> np.uint32(32 - d))).astype(np.uint32)


def _threefry2x32_pair(key, x0, x1):
    x0 = x0.astype(np.uint32).copy()
    x1 = x1.astype(np.uint32).copy()
    ks = [np.uint32(key[0]), np.uint32(key[1])]
    ks.append(np.uint32(ks[0] ^ ks[1] ^ np.uint32(0x1BD11BDA)))
    x0 = (x0 + ks[0]).astype(np.uint32)
    x1 = (x1 + ks[1]).astype(np.uint32)
    rotations = [[13, 15, 26, 6], [17, 29, 16, 24]]
    for i in range(5):
        for r in rotations[i % 2]:
            x0 = (x0 + x1).astype(np.uint32)
            x1 = _rotl(x1, r)
            x1 = x1 ^ x0
        x0 = (x0 + ks[(i + 1) % 3]).astype(np.uint32)
        x1 = (x1 + ks[(i + 2) % 3] + np.uint32(i + 1)).astype(np.uint32)
    return x0, x1


def _get_offsets() -> np.ndarray:
    key = np.array([0, 42], dtype=np.uint32)
    b1, b2 = _threefry2x32_pair(
        key, np.zeros(2, np.uint32), np.arange(2, dtype=np.uint32)
    )
    k2 = np.array([b1[1], b2[1]], dtype=np.uint32)
    c1, c2 = _threefry2x32_pair(
        k2, np.zeros(32, np.uint32), np.arange(32, dtype=np.uint32)
    )
    bits = (c1 ^ c2).reshape(_SOURCES, _BATCH)
    return (bits % np.uint32(_SHIFT)).astype(np.int32)


def _select_const(wid, values):
    acc = jnp.int32(values[0])
    for w in range(1, len(values)):
        acc = lax.select(wid == w, jnp.int32(values[w]), acc)
    return acc


def _make_shift_kernel(offs):
    mesh = plsc.VectorSubcoreMesh(
        core_axis_name="c", subcore_axis_name="s", num_cores=_NC
    )
    scratch = [
        pltpu.VMEM((_IBROWS, _LANES), jnp.float32),
        pltpu.VMEM((_IBROWS, _LANES), jnp.float32),
        pltpu.VMEM((_CROWS, _LANES), jnp.float32),
        pltpu.VMEM((_CROWS, _LANES), jnp.float32),
    ]
    scratch += [pltpu.SemaphoreType.DMA] * 4

    arows = [
        w * _IN_R + 2 * ((o // _LANES) - (o // _LANES) % 4)
        for w, o in enumerate(offs)
    ]
    mqs = [(o // _LANES) % 4 for o in offs]
    rs = [o % _LANES for o in offs]

    @functools.partial(
        pl.kernel,
        out_type=jax.ShapeDtypeStruct((_NW * _OUT_R, _LANES), jnp.float32),
        mesh=mesh,
        scratch_types=scratch,
        compiler_params=pltpu.CompilerParams(needs_layout_passes=False),
    )
    def shift_kernel(wav_hbm, out_hbm, ib0, ib1, ob0, ob1, si0, si1, so0, so1):
        wav2d = wav_hbm
        out2d = out_hbm
        ibufs, obufs = [ib0, ib1], [ob0, ob1]
        in_sems, out_sems = [si0, si1], [so0, so1]
        wid = lax.axis_index("s") * _NC + lax.axis_index("c")
        arow = pl.multiple_of(_select_const(wid, arows), 8)
        obrow = pl.multiple_of(wid * _OUT_R, 8)
        mq = _select_const(wid, mqs)
        r = _select_const(wid, rs)
        lane = jnp.arange(16, dtype=jnp.int32)
        rvec = lane + r

        def in_copy(k):
            return pltpu.make_async_copy(
                wav2d.at[pl.ds(arow + k * _CROWS, _IBROWS), :],
                ibufs[k % 2],
                in_sems[k % 2],
            )

        def out_copy(k):
            return pltpu.make_async_copy(
                obufs[k % 2],
                out2d.at[pl.ds(obrow + k * _CROWS, _CROWS), :],
                out_sems[k % 2],
            )

        carry2s = []
        cols = []
        for u in range(8):
            svec = rvec + u * 16
            carry2s.append((svec >> 7) << 1)
            cols.append(svec & 127)

        def shift_chunk(k):
            ib, ob = ibufs[k % 2], obufs[k % 2]

            @plsc.parallel_loop(0, _CHUNK // 16, step=8, unroll=1)
            def body(w0):
                j = w0 >> 4
                c = (w0 >> 3) & 1
                rowbase = 2 * (j + mq) + c
                orow = w0 >> 3
                for u in range(8):
                    vec = plsc.load_gather(
                        ib, [rowbase + carry2s[u], cols[u]]
                    )
                    ob[orow, pl.ds(u * 16, 16)] = vec

        in_copy(0).start()
        in_copy(1).start()
        for k in range(_NCHUNK):
            in_copy(k).wait()
            if k >= 2:
                out_copy(k - 2).wait()
            shift_chunk(k)
            out_copy(k).start()
            if k + 2 < _NCHUNK:
                in_copy(k + 2).start()
        out_copy(_NCHUNK - 2).wait()
        out_copy(_NCHUNK - 1).wait()

    return shift_kernel


def kernel(wav):
    offs_np = _get_offsets()
    offs = [int(offs_np[w // _BATCH, w % _BATCH]) for w in range(_NW)]
    wav2 = (
        wav.reshape(_SOURCES, _BATCH, _IN_T, _LANES, _CHANNELS)
        .transpose(0, 1, 2, 4, 3)
        .reshape(_NW * _IN_R, _LANES)
    )
    out2 = _make_shift_kernel(offs)(wav2)
    out = (
        out2.reshape(_SOURCES, _BATCH, _OUT_T, _CHANNELS, _LANES)
        .transpose(0, 1, 2, 4, 3)
        .reshape(_SOURCES, _BATCH, _OUT_LEN, _CHANNELS)
    )
    offsets = jnp.asarray(offs_np)
    return (out, offsets)


# speedup vs baseline: 1.0906x; 1.0423x over previous
import functools

import jax
import jax.numpy as jnp
import numpy as np
from jax import lax
from jax.experimental import pallas as pl
from jax.experimental.pallas import tpu as pltpu
from jax.experimental.pallas import tpu_sc as plsc

_SHIFT = 8192
_SOURCES = 2
_BATCH = 16
_LENGTH = 163840
_CHANNELS = 2
_OUT_LEN = _LENGTH - _SHIFT

_NC = 2
_NS = 16
_NW = _NC * _NS

_LANES = 128
_IN_T = _LENGTH // _LANES
_OUT_T = _OUT_LEN // _LANES
_IN_R = 2 * _IN_T
_OUT_R = 2 * _OUT_T
_CTILES = 76
_CROWS = 2 * _CTILES
_CHUNK = _CROWS * _LANES
_NCHUNK = _OUT_T // _CTILES
_IBROWS = _CROWS + 16


def _rotl(x, d):
    return ((x << np.uint32(d)) | (x >> np.uint32(32 - d))).astype(np.uint32)


def _threefry2x32_pair(key, x0, x1):
    x0 = x0.astype(np.uint32).copy()
    x1 = x1.astype(np.uint32).copy()
    ks = [np.uint32(key[0]), np.uint32(key[1])]
    ks.append(np.uint32(ks[0] ^ ks[1] ^ np.uint32(0x1BD11BDA)))
    x0 = (x0 + ks[0]).astype(np.uint32)
    x1 = (x1 + ks[1]).astype(np.uint32)
    rotations = [[13, 15, 26, 6], [17, 29, 16, 24]]
    for i in range(5):
        for r in rotations[i % 2]:
            x0 = (x0 + x1).astype(np.uint32)
            x1 = _rotl(x1, r)
            x1 = x1 ^ x0
        x0 = (x0 + ks[(i + 1) % 3]).astype(np.uint32)
        x1 = (x1 + ks[(i + 2) % 3] + np.uint32(i + 1)).astype(np.uint32)
    return x0, x1


def _get_offsets() -> np.ndarray:
    key = np.array([0, 42], dtype=np.uint32)
    b1, b2 = _threefry2x32_pair(
        key, np.zeros(2, np.uint32), np.arange(2, dtype=np.uint32)
    )
    k2 = np.array([b1[1], b2[1]], dtype=np.uint32)
    c1, c2 = _threefry2x32_pair(
        k2, np.zeros(32, np.uint32), np.arange(32, dtype=np.uint32)
    )
    bits = (c1 ^ c2).reshape(_SOURCES, _BATCH)
    return (bits % np.uint32(_SHIFT)).astype(np.int32)


def _select_const(wid, values):
    acc = jnp.int32(values[0])
    for w in range(1, len(values)):
        acc = lax.select(wid == w, jnp.int32(values[w]), acc)
    return acc


def _make_shift_kernel(offs):
    mesh = plsc.VectorSubcoreMesh(
        core_axis_name="c", subcore_axis_name="s", num_cores=_NC
    )
    scratch = [
        pltpu.VMEM((_IBROWS, _LANES), jnp.float32),
        pltpu.VMEM((_IBROWS, _LANES), jnp.float32),
        pltpu.VMEM((_CROWS, _LANES), jnp.float32),
        pltpu.VMEM((_CROWS, _LANES), jnp.float32),
    ]
    scratch += [pltpu.SemaphoreType.DMA] * 4

    arows = [
        w * _IN_R + 2 * ((o // _LANES) - (o // _LANES) % 4)
        for w, o in enumerate(offs)
    ]
    mqs = [(o // _LANES) % 4 for o in offs]
    rs = [o % _LANES for o in offs]

    @functools.partial(
        pl.kernel,
        out_type=jax.ShapeDtypeStruct((_NW * _OUT_R, _LANES), jnp.float32),
        mesh=mesh,
        scratch_types=scratch,
        compiler_params=pltpu.CompilerParams(needs_layout_passes=False),
    )
    def shift_kernel(wav_hbm, out_hbm, ib0, ib1, ob0, ob1, si0, si1, so0, so1):
        wav2d = wav_hbm
        out2d = out_hbm
        ibufs, obufs = [ib0, ib1], [ob0, ob1]
        in_sems, out_sems = [si0, si1], [so0, so1]
        wid = lax.axis_index("s") * _NC + lax.axis_index("c")
        arow = pl.multiple_of(_select_const(wid, arows), 8)
        obrow = pl.multiple_of(wid * _OUT_R, 8)
        mq = _select_const(wid, mqs)
        r = _select_const(wid, rs)
        lane = jnp.arange(16, dtype=jnp.int32)
        rvec = lane + r

        def in_copy(k):
            return pltpu.make_async_copy(
                wav2d.at[pl.ds(arow + k * _CROWS, _IBROWS), :],
                ibufs[k % 2],
                in_sems[k % 2],
            )

        def out_copy(k):
            return pltpu.make_async_copy(
                obufs[k % 2],
                out2d.at[pl.ds(obrow + k * _CROWS, _CROWS), :],
                out_sems[k % 2],
            )

        def shift_chunk(k):
            ib, ob = ibufs[k % 2], obufs[k % 2]

            @plsc.parallel_loop(0, _CHUNK // 16, unroll=8)
            def body(w):
                j = w >> 4
                c = (w >> 3) & 1
                svec = rvec + (w & 7) * 16
                carry = svec >> 7
                col = svec & 127
                row = (2 * (j + mq) + c) + (carry << 1)
                vec = plsc.load_gather(ib, [row, col])
                ob[w >> 3, pl.ds(pl.multiple_of((w & 7) * 16, 8), 16)] = vec

        in_copy(0).start()
        in_copy(1).start()
        for k in range(_NCHUNK):
            in_copy(k).wait()
            if k >= 2:
                out_copy(k - 2).wait()
            shift_chunk(k)
            out_copy(k).start()
            if k + 2 < _NCHUNK:
                in_copy(k + 2).start()
        out_copy(_NCHUNK - 2).wait()
        out_copy(_NCHUNK - 1).wait()

    return shift_kernel


def kernel(wav):
    offs_np = _get_offsets()
    offs = [int(offs_np[w // _BATCH, w % _BATCH]) for w in range(_NW)]
    wav2 = (
        wav.reshape(_SOURCES, _BATCH, _IN_T, _LANES, _CHANNELS)
        .transpose(0, 1, 2, 4, 3)
        .reshape(_NW * _IN_R, _LANES)
    )
    out2 = _make_shift_kernel(offs)(wav2)
    out = (
        out2.reshape(_SOURCES, _BATCH, _OUT_T, _CHANNELS, _LANES)
        .transpose(0, 1, 2, 4, 3)
        .reshape(_SOURCES, _BATCH, _OUT_LEN, _CHANNELS)
    )
    offsets = jnp.asarray(offs_np)
    return (out, offsets)


# speedup vs baseline: 1.0916x; 1.0009x over previous
import functools

import jax
import jax.numpy as jnp
import numpy as np
from jax import lax
from jax.experimental import pallas as pl
from jax.experimental.pallas import tpu as pltpu
from jax.experimental.pallas import tpu_sc as plsc

_SHIFT = 8192
_SOURCES = 2
_BATCH = 16
_LENGTH = 163840
_CHANNELS = 2
_OUT_LEN = _LENGTH - _SHIFT

_NC = 2
_NS = 16
_NW = _NC * _NS

_LANES = 128
_IN_T = _LENGTH // _LANES
_OUT_T = _OUT_LEN // _LANES
_IN_R = 2 * _IN_T
_OUT_R = 2 * _OUT_T
_CTILES = 76
_CROWS = 2 * _CTILES
_CHUNK = _CROWS * _LANES
_NCHUNK = _OUT_T // _CTILES
_IBROWS = _CROWS + 16


def _rotl(x, d):
    return ((x << np.uint32(d)) | (x >> np.uint32(32 - d))).astype(np.uint32)


def _threefry2x32_pair(key, x0, x1):
    x0 = x0.astype(np.uint32).copy()
    x1 = x1.astype(np.uint32).copy()
    ks = [np.uint32(key[0]), np.uint32(key[1])]
    ks.append(np.uint32(ks[0] ^ ks[1] ^ np.uint32(0x1BD11BDA)))
    x0 = (x0 + ks[0]).astype(np.uint32)
    x1 = (x1 + ks[1]).astype(np.uint32)
    rotations = [[13, 15, 26, 6], [17, 29, 16, 24]]
    for i in range(5):
        for r in rotations[i % 2]:
            x0 = (x0 + x1).astype(np.uint32)
            x1 = _rotl(x1, r)
            x1 = x1 ^ x0
        x0 = (x0 + ks[(i + 1) % 3]).astype(np.uint32)
        x1 = (x1 + ks[(i + 2) % 3] + np.uint32(i + 1)).astype(np.uint32)
    return x0, x1


def _get_offsets() -> np.ndarray:
    key = np.array([0, 42], dtype=np.uint32)
    b1, b2 = _threefry2x32_pair(
        key, np.zeros(2, np.uint32), np.arange(2, dtype=np.uint32)
    )
    k2 = np.array([b1[1], b2[1]], dtype=np.uint32)
    c1, c2 = _threefry2x32_pair(
        k2, np.zeros(32, np.uint32), np.arange(32, dtype=np.uint32)
    )
    bits = (c1 ^ c2).reshape(_SOURCES, _BATCH)
    return (bits % np.uint32(_SHIFT)).astype(np.int32)


def _select_const(wid, values):
    acc = jnp.int32(values[0])
    for w in range(1, len(values)):
        acc = lax.select(wid == w, jnp.int32(values[w]), acc)
    return acc


def _make_shift_kernel(offs):
    mesh = plsc.VectorSubcoreMesh(
        core_axis_name="c", subcore_axis_name="s", num_cores=_NC
    )
    scratch = [
        pltpu.VMEM((_IBROWS, _LANES), jnp.float32),
        pltpu.VMEM((_IBROWS, _LANES), jnp.float32),
        pltpu.VMEM((_CROWS, _LANES), jnp.float32),
        pltpu.VMEM((_CROWS, _LANES), jnp.float32),
    ]
    scratch += [pltpu.SemaphoreType.DMA] * 4

    arows = [
        w * _IN_R + 2 * ((o // _LANES) - (o // _LANES) % 4)
        for w, o in enumerate(offs)
    ]
    mqs = [(o // _LANES) % 4 for o in offs]
    rs = [o % _LANES for o in offs]

    @functools.partial(
        pl.kernel,
        out_type=jax.ShapeDtypeStruct((_NW * _OUT_R, _LANES), jnp.float32),
        mesh=mesh,
        scratch_types=scratch,
        compiler_params=pltpu.CompilerParams(needs_layout_passes=False),
    )
    def shift_kernel(wav_hbm, out_hbm, ib0, ib1, ob0, ob1, si0, si1, so0, so1):
        wav2d = wav_hbm
        out2d = out_hbm
        ibufs, obufs = [ib0, ib1], [ob0, ob1]
        in_sems, out_sems = [si0, si1], [so0, so1]
        wid = lax.axis_index("s") * _NC + lax.axis_index("c")
        arow = pl.multiple_of(_select_const(wid, arows), 8)
        obrow = pl.multiple_of(wid * _OUT_R, 8)
        mq = _select_const(wid, mqs)
        r = _select_const(wid, rs)
        lane = jnp.arange(16, dtype=jnp.int32)
        rvec = lane + r

        def in_copy(k):
            return pltpu.make_async_copy(
                wav2d.at[pl.ds(arow + k * _CROWS, _IBROWS), :],
                ibufs[k % 2],
                in_sems[k % 2],
            )

        def out_copy(k):
            return pltpu.make_async_copy(
                obufs[k % 2],
                out2d.at[pl.ds(obrow + k * _CROWS, _CROWS), :],
                out_sems[k % 2],
            )

        def shift_chunk(k):
            ib, ob = ibufs[k % 2], obufs[k % 2]

            @plsc.parallel_loop(0, _CHUNK // 16, unroll=16)
            def body(w):
                j = w >> 4
                c = (w >> 3) & 1
                svec = rvec + (w & 7) * 16
                carry = svec >> 7
                col = svec & 127
                row = (2 * (j + mq) + c) + (carry << 1)
                vec = plsc.load_gather(ib, [row, col])
                ob[w >> 3, pl.ds(pl.multiple_of((w & 7) * 16, 8), 16)] = vec

        in_copy(0).start()
        in_copy(1).start()
        for k in range(_NCHUNK):
            in_copy(k).wait()
            if k >= 2:
                out_copy(k - 2).wait()
            shift_chunk(k)
            out_copy(k).start()
            if k + 2 < _NCHUNK:
                in_copy(k + 2).start()
        out_copy(_NCHUNK - 2).wait()
        out_copy(_NCHUNK - 1).wait()

    return shift_kernel


def kernel(wav):
    offs_np = _get_offsets()
    offs = [int(offs_np[w // _BATCH, w % _BATCH]) for w in range(_NW)]
    wav2 = (
        wav.reshape(_SOURCES, _BATCH, _IN_T, _LANES, _CHANNELS)
        .transpose(0, 1, 2, 4, 3)
        .reshape(_NW * _IN_R, _LANES)
    )
    out2 = _make_shift_kernel(offs)(wav2)
    out = (
        out2.reshape(_SOURCES, _BATCH, _OUT_T, _CHANNELS, _LANES)
        .transpose(0, 1, 2, 4, 3)
        .reshape(_SOURCES, _BATCH, _OUT_LEN, _CHANNELS)
    )
    offsets = jnp.asarray(offs_np)
    return (out, offsets)
